# Initial kernel scaffold; baseline (speedup 1.0000x reference)
import jax
import jax.numpy as jnp
from jax.experimental import pallas as pl


def kernel(x):
    raise NotImplementedError("write your pallas kernel here")


# speedup vs baseline: 1.0307x; 1.0307x over previous
import jax
import jax.numpy as jnp
from jax.experimental import pallas as pl
from jax.experimental.pallas import tpu as pltpu


def _outer_kernel(x_ref, erep_ref, o_ref):
    x = x_ref[0]
    n = x.shape[1]
    nn = o_ref.shape[2]
    x2 = jnp.concatenate([x, x], axis=1)
    xj = pltpu.repeat(x2, nn // (2 * n), axis=1)
    xi = jnp.dot(x, erep_ref[...], preferred_element_type=jnp.float32)
    o_ref[0] = (xi * xj).astype(o_ref.dtype)


def kernel(x):
    B, C, N = x.shape
    NN = N * N
    itemsize = x.dtype.itemsize

    target = 2 * 1024 * 1024
    tc = C
    if C * NN * itemsize > target:
        cap = max(8, target // (NN * itemsize))
        tc = (min(C, cap) // 8) * 8
        while tc > 8 and C % tc:
            tc -= 8

    e_rep = jnp.repeat(jnp.eye(N, dtype=x.dtype), N, axis=1)

    flat = pl.pallas_call(
        _outer_kernel,
        out_shape=jax.ShapeDtypeStruct((B, C, NN), x.dtype),
        grid=(B, C // tc),
        in_specs=[
            pl.BlockSpec((1, tc, N), lambda b, c: (b, c, 0)),
            pl.BlockSpec((N, NN), lambda b, c: (0, 0)),
        ],
        out_specs=pl.BlockSpec((1, tc, NN), lambda b, c: (b, c, 0)),
        compiler_params=pltpu.CompilerParams(
            dimension_semantics=("parallel", "parallel"),
            vmem_limit_bytes=64 * 1024 * 1024,
        ),
        cost_estimate=pl.CostEstimate(
            flops=B * C * NN + 2 * B * C * N * NN,
            transcendentals=0,
            bytes_accessed=(B * C * N + N * NN + B * C * NN) * itemsize,
        ),
    )(x, e_rep)

    return flat.reshape(B, C, N, N)
